# Initial kernel scaffold; baseline (speedup 1.0000x reference)
import jax
import jax.numpy as jnp
from jax.experimental import pallas as pl


def kernel(node_pos, edge_index, batch_ids):
    raise NotImplementedError("write your pallas kernel here")


# speedup vs baseline: 124.3271x; 124.3271x over previous
import functools

import jax
import jax.numpy as jnp
from jax import lax
from jax.experimental import pallas as pl
from jax.experimental.pallas import tpu as pltpu
from jax.experimental.pallas import tpu_sc as plsc

N_NODES = 100000
N_EDGES = 6400000
NUM_GRAPHS = 1024

NC = 2
NS = 16
L = 16
NW = NC * NS
EPW = N_EDGES // NW
CHUNK = 2000
NCHUNK = EPW // CHUNK
CVECS = CHUNK // L


def _edge_var_sc(xp, yp, zp, bp, src, dst):
    mesh = plsc.VectorSubcoreMesh(
        core_axis_name="c", subcore_axis_name="s", num_cores=NC, num_subcores=NS
    )

    @functools.partial(
        pl.kernel,
        out_type=[
            jax.ShapeDtypeStruct((NW, NUM_GRAPHS), jnp.float32),
            jax.ShapeDtypeStruct((NW, NUM_GRAPHS), jnp.float32),
        ],
        mesh=mesh,
        scratch_types=[
            pltpu.VMEM_SHARED((N_NODES,), jnp.float32),
            pltpu.VMEM_SHARED((N_NODES,), jnp.float32),
            pltpu.VMEM_SHARED((N_NODES,), jnp.float32),
            pltpu.VMEM_SHARED((N_NODES,), jnp.int32),
            pltpu.VMEM((CHUNK,), jnp.int32),
            pltpu.VMEM((CHUNK,), jnp.int32),
            pltpu.VMEM((CHUNK,), jnp.float32),
            pltpu.VMEM((CHUNK,), jnp.float32),
            pltpu.VMEM((CHUNK,), jnp.float32),
            pltpu.VMEM((CHUNK,), jnp.int32),
            pltpu.VMEM((CHUNK,), jnp.float32),
            pltpu.VMEM((CHUNK,), jnp.float32),
            pltpu.VMEM((CHUNK,), jnp.float32),
            pltpu.VMEM((NUM_GRAPHS,), jnp.float32),
            pltpu.VMEM((NUM_GRAPHS,), jnp.float32),
            pltpu.SemaphoreType.DMA,
        ],
        compiler_params=pltpu.CompilerParams(needs_layout_passes=False),
    )
    def body(x_hbm, y_hbm, z_hbm, b_hbm, src_hbm, dst_hbm, sums_out, cnts_out,
             xs, ys, zs, bs, sidx, didx, gsx, gsy, gsz, gsb, gdx, gdy, gdz,
             lsum, lcnt, sem):
        cid = lax.axis_index("c")
        sid = lax.axis_index("s")
        wid = cid * NS + sid

        @pl.when(sid == 0)
        def _():
            pltpu.sync_copy(x_hbm, xs)
            pltpu.sync_copy(y_hbm, ys)
            pltpu.sync_copy(z_hbm, zs)
            pltpu.sync_copy(b_hbm, bs)

        def zbody(i, _):
            off = pl.multiple_of(i * L, L)
            lsum[pl.ds(off, L)] = jnp.zeros((L,), jnp.float32)
            lcnt[pl.ds(off, L)] = jnp.zeros((L,), jnp.float32)
            return 0

        lax.fori_loop(0, NUM_GRAPHS // L, zbody, 0)
        plsc.subcore_barrier()

        ones = jnp.ones((L,), jnp.float32)

        def vec_body(vi, _):
            off = pl.multiple_of(vi * L, L)
            sl = pl.ds(off, L)
            sx = gsx[sl]
            sy = gsy[sl]
            sz = gsz[sl]
            dx = gdx[sl]
            dy = gdy[sl]
            dz = gdz[sl]
            ex = dx - sx
            ey = dy - sy
            ez = dz - sz
            s = ex * ex + ey * ey + ez * ez + jnp.float32(1e-12)
            bits = plsc.bitcast(s, jnp.int32)
            bits = jnp.int32(0x5F3759DF) - lax.shift_right_logical(bits, 1)
            y = plsc.bitcast(bits, jnp.float32)
            half = s * jnp.float32(0.5)
            for _ in range(3):
                y = y * (jnp.float32(1.5) - half * y * y)
            eu = s * y
            d = eu - jnp.float32(1.0)
            var = d * d
            bidx = gsb[sl]
            plsc.addupdate_scatter(lsum, [bidx], var)
            plsc.addupdate_scatter(lcnt, [bidx], ones)
            return 0

        def chunk_body(ci, _):
            base = pl.multiple_of(wid * EPW + ci * CHUNK, 8)
            pltpu.sync_copy(src_hbm.at[pl.ds(base, CHUNK)], sidx)
            pltpu.sync_copy(dst_hbm.at[pl.ds(base, CHUNK)], didx)
            cps = [
                pltpu.async_copy(xs.at[sidx], gsx, sem),
                pltpu.async_copy(ys.at[sidx], gsy, sem),
                pltpu.async_copy(zs.at[sidx], gsz, sem),
                pltpu.async_copy(bs.at[sidx], gsb, sem),
                pltpu.async_copy(xs.at[didx], gdx, sem),
                pltpu.async_copy(ys.at[didx], gdy, sem),
                pltpu.async_copy(zs.at[didx], gdz, sem),
            ]
            for cp in cps:
                cp.wait()
            lax.fori_loop(0, CVECS, vec_body, 0)
            return 0

        lax.fori_loop(0, NCHUNK, chunk_body, 0)

        pltpu.sync_copy(lsum, sums_out.at[wid])
        pltpu.sync_copy(lcnt, cnts_out.at[wid])

    return body(xp, yp, zp, bp, src, dst)


def _finalize_tc(sums_p, cnts_p):
    def tc_body(s_ref, c_ref, o_ref):
        s = jnp.sum(s_ref[...], axis=0)
        c = jnp.sum(c_ref[...], axis=0)
        gv = s / jnp.maximum(c, 1.0)
        o_ref[...] = (jnp.sum(gv) / jnp.float32(NUM_GRAPHS)).reshape(1, 1)

    out = pl.pallas_call(
        tc_body,
        out_shape=jax.ShapeDtypeStruct((1, 1), jnp.float32),
    )(sums_p, cnts_p)
    return out[0, 0]


def kernel(node_pos, edge_index, batch_ids):
    bi = batch_ids.astype(jnp.int32)
    xp = node_pos[:, 0]
    yp = node_pos[:, 1]
    zp = node_pos[:, 2]
    ei = edge_index.astype(jnp.int32)
    src = ei[0]
    dst = ei[1]
    sums_p, cnts_p = _edge_var_sc(xp, yp, zp, bi, src, dst)
    return _finalize_tc(sums_p, cnts_p)
